# Initial kernel scaffold; baseline (speedup 1.0000x reference)
import jax
import jax.numpy as jnp
from jax.experimental import pallas as pl


def kernel(xyz, features, mlp_params, final_params):
    raise NotImplementedError("write your pallas kernel here")


# speedup vs baseline: 7.2008x; 7.2008x over previous
import functools

import jax
import jax.numpy as jnp
from jax import lax
from jax.experimental import pallas as pl
from jax.experimental.pallas import tpu as pltpu
from jax.experimental.pallas import tpu_sc as plsc

_NPOINT = 1024
_RADIUS_LIST = [0.1, 0.2, 0.4]
_NSAMPLE_LIST = [16, 32, 64]
_B, _N = 8, 4096
_CPAD = 128


def _fps_kernel(xyz_ref, out_ref):
    x = xyz_ref[:, 0:8]
    y = xyz_ref[:, 8:16]
    z = xyz_ref[:, 16:24]
    iota_n = lax.broadcasted_iota(jnp.int32, (_N, 8), 0)

    def body(i, carry):
        dist, far = carry
        oh = (iota_n == far).astype(jnp.float32)
        cx = jnp.sum(oh * x, axis=0, keepdims=True)
        cy = jnp.sum(oh * y, axis=0, keepdims=True)
        cz = jnp.sum(oh * z, axis=0, keepdims=True)
        out_ref[pl.ds(i, 1), 0:8] = cx
        out_ref[pl.ds(i, 1), 8:16] = cy
        out_ref[pl.ds(i, 1), 16:24] = cz
        dx = x - cx
        dy = y - cy
        dz = z - cz
        d = dx * dx + dy * dy + dz * dz
        dist = jnp.minimum(dist, d)
        m = jnp.max(dist, axis=0, keepdims=True)
        far = jnp.min(jnp.where(dist == m, iota_n, _N), axis=0, keepdims=True)
        return dist, far

    dist0 = jnp.full((_N, 8), 1e10, jnp.float32)
    far0 = jnp.zeros((1, 8), jnp.int32)
    lax.fori_loop(0, _NPOINT, body, (dist0, far0))


def _run_fps(xyz):
    xt = jnp.transpose(xyz, (1, 2, 0)).reshape(_N, 24)
    out = pl.pallas_call(
        _fps_kernel,
        out_shape=jax.ShapeDtypeStruct((_NPOINT, 24), jnp.float32),
    )(xt)
    return out.reshape(_NPOINT, 3, _B).transpose(2, 0, 1)


def _ballq_kernel(d_ref, o1_ref, o2_ref, o3_ref, *, st):
    d = d_ref[0]
    iota = lax.broadcasted_iota(jnp.int32, (st, _N), 1)
    for r, k, oref in zip(_RADIUS_LIST, _NSAMPLE_LIST, (o1_ref, o2_ref, o3_ref)):
        mio = jnp.where(d > r * r, _N, iota)
        first = None
        for j in range(k):
            cur = jnp.min(mio, axis=1, keepdims=True)
            if j == 0:
                first = cur
                oref[0, :, 0:1] = cur
            else:
                oref[0, :, j:j + 1] = jnp.where(cur == _N, first, cur)
            mio = jnp.where(mio == cur, _N, mio)


def _run_ballq(xyz, new_xyz, st=128):
    sqrdists = (
        jnp.sum(new_xyz ** 2, -1)[:, :, None]
        + jnp.sum(xyz ** 2, -1)[:, None, :]
        - 2.0 * jnp.einsum("bsc,bnc->bsn", new_xyz, xyz)
    )
    grid = (_B, _NPOINT // st)
    outs = [jax.ShapeDtypeStruct((_B, _NPOINT, k), jnp.int32) for k in _NSAMPLE_LIST]
    return pl.pallas_call(
        functools.partial(_ballq_kernel, st=st),
        grid=grid,
        in_specs=[
            pl.BlockSpec((1, st, _N), lambda b, s: (b, s, 0)),
        ],
        out_specs=[
            pl.BlockSpec((1, st, k), lambda b, s: (b, s, 0)) for k in _NSAMPLE_LIST
        ],
        out_shape=outs,
    )(sqrdists)


def _sc_gather(table, gidx):
    info = plsc.get_sparse_core_info()
    nw = info.num_cores * info.num_subcores
    m = gidx.shape[0]
    per_w = m // nw
    ch = 512
    n_ch = per_w // ch
    assert per_w % ch == 0
    mesh = plsc.VectorSubcoreMesh(core_axis_name="c", subcore_axis_name="s")

    @functools.partial(
        pl.kernel,
        mesh=mesh,
        out_type=jax.ShapeDtypeStruct((m, _CPAD), jnp.float32),
        scratch_types=[
            pltpu.VMEM((ch,), jnp.int32),
            pltpu.VMEM((ch, _CPAD), jnp.float32),
            pltpu.SemaphoreType.DMA,
        ],
    )
    def k(table_hbm, idx_hbm, out_hbm, idx_v, rows_v, sem):
        wid = lax.axis_index("s") * info.num_cores + lax.axis_index("c")
        base = wid * per_w
        for c in range(n_ch):
            off = base + c * ch
            pltpu.sync_copy(idx_hbm.at[pl.ds(off, ch)], idx_v)
            pltpu.async_copy(table_hbm.at[idx_v], rows_v, sem).wait()
            pltpu.sync_copy(rows_v, out_hbm.at[pl.ds(off, ch)])

    return k(table, gidx)


def _mm_kernel(x_ref, w_ref, z_ref, st_ref, *, stt, kk, cc, oc):
    b = pl.program_id(0)
    s = pl.program_id(1)

    @pl.when(jnp.logical_and(b == 0, s == 0))
    def _():
        st_ref[...] = jnp.zeros((8, oc), jnp.float32)

    x = x_ref[0]
    z = jnp.dot(
        x.reshape(stt * kk, cc), w_ref[...], preferred_element_type=jnp.float32
    )
    z_ref[0] = z.reshape(stt, kk, oc)
    st_ref[0:1, :] += jnp.sum(z, axis=0, keepdims=True)
    st_ref[1:2, :] += jnp.sum(z * z, axis=0, keepdims=True)


def _mm_sub_kernel(x_ref, sub_ref, w_ref, z_ref, st_ref, *, stt, kk, cc, oc):
    b = pl.program_id(0)
    s = pl.program_id(1)

    @pl.when(jnp.logical_and(b == 0, s == 0))
    def _():
        st_ref[...] = jnp.zeros((8, oc), jnp.float32)

    x = x_ref[0] - sub_ref[0][:, None, :]
    z = jnp.dot(
        x.reshape(stt * kk, cc), w_ref[...], preferred_element_type=jnp.float32
    )
    z_ref[0] = z.reshape(stt, kk, oc)
    st_ref[0:1, :] += jnp.sum(z, axis=0, keepdims=True)
    st_ref[1:2, :] += jnp.sum(z * z, axis=0, keepdims=True)


def _run_mm(x, wt, sub=None):
    bb, ss, kk, cc = x.shape
    oc = wt.shape[1]
    stt = 128
    while stt * kk * max(cc, oc) > 1 << 19:
        stt //= 2
    grid = (bb, ss // stt)
    outs = [
        jax.ShapeDtypeStruct((bb, ss, kk, oc), jnp.float32),
        jax.ShapeDtypeStruct((8, oc), jnp.float32),
    ]
    out_specs = [
        pl.BlockSpec((1, stt, kk, oc), lambda b, s: (b, s, 0, 0)),
        pl.BlockSpec((8, oc), lambda b, s: (0, 0)),
    ]
    if sub is None:
        return pl.pallas_call(
            functools.partial(_mm_kernel, stt=stt, kk=kk, cc=cc, oc=oc),
            grid=grid,
            in_specs=[
                pl.BlockSpec((1, stt, kk, cc), lambda b, s: (b, s, 0, 0)),
                pl.BlockSpec((cc, oc), lambda b, s: (0, 0)),
            ],
            out_specs=out_specs,
            out_shape=outs,
        )(x, wt)
    return pl.pallas_call(
        functools.partial(_mm_sub_kernel, stt=stt, kk=kk, cc=cc, oc=oc),
        grid=grid,
        in_specs=[
            pl.BlockSpec((1, stt, kk, cc), lambda b, s: (b, s, 0, 0)),
            pl.BlockSpec((1, stt, cc), lambda b, s: (b, s, 0)),
            pl.BlockSpec((cc, oc), lambda b, s: (0, 0)),
        ],
        out_specs=out_specs,
        out_shape=outs,
    )(x, sub, wt)


def _bn_kernel(z_ref, st_ref, g_ref, bt_ref, a_ref, o_ref, *, cnt, pool):
    mean = st_ref[0:1, :] / cnt
    var = st_ref[1:2, :] / cnt - mean * mean
    inv = 1.0 / jnp.sqrt(var + 1e-5)
    z = z_ref[0]
    y = (z - mean[None]) * inv[None] * g_ref[0][None, None, :] + bt_ref[0][None, None, :]
    y = jnp.where(y > 0, y, a_ref[0, 0] * y)
    if pool:
        o_ref[0] = jnp.max(y, axis=1)
    else:
        o_ref[0] = y


def _run_bn(z, stats, gamma, beta, a, cnt, pool):
    bb, ss, kk, oc = z.shape
    stt = 128
    while stt * kk * oc > 1 << 19:
        stt //= 2
    grid = (bb, ss // stt)
    if pool:
        out_shape = jax.ShapeDtypeStruct((bb, ss, oc), jnp.float32)
        out_spec = pl.BlockSpec((1, stt, oc), lambda b, s: (b, s, 0))
    else:
        out_shape = jax.ShapeDtypeStruct((bb, ss, kk, oc), jnp.float32)
        out_spec = pl.BlockSpec((1, stt, kk, oc), lambda b, s: (b, s, 0, 0))
    return pl.pallas_call(
        functools.partial(_bn_kernel, cnt=float(cnt), pool=pool),
        grid=grid,
        in_specs=[
            pl.BlockSpec((1, stt, kk, oc), lambda b, s: (b, s, 0, 0)),
            pl.BlockSpec((8, oc), lambda b, s: (0, 0)),
            pl.BlockSpec((1, oc), lambda b, s: (0, 0)),
            pl.BlockSpec((1, oc), lambda b, s: (0, 0)),
            pl.BlockSpec((1, 1), lambda b, s: (0, 0)),
        ],
        out_specs=out_spec,
        out_shape=out_shape,
    )(z, stats, gamma.reshape(1, oc), beta.reshape(1, oc), a.reshape(1, 1))


def kernel(xyz, features, mlp_params, final_params):
    new_xyz = _run_fps(xyz)
    i1, i2, i3 = _run_ballq(xyz, new_xyz)
    gidx = jnp.concatenate([i1, i2, i3], axis=2)
    gidx = gidx + (jnp.arange(_B, dtype=jnp.int32) * _N)[:, None, None]
    table = jnp.concatenate(
        [features, xyz, jnp.zeros((_B, _N, _CPAD - 9), jnp.float32)], axis=-1
    ).reshape(_B * _N, _CPAD)
    g = _sc_gather(table, gidx.reshape(-1)).reshape(_B, _NPOINT, 112, _CPAD)
    sub = jnp.concatenate(
        [
            jnp.zeros((_B, _NPOINT, 6), jnp.float32),
            new_xyz,
            jnp.zeros((_B, _NPOINT, _CPAD - 9), jnp.float32),
        ],
        axis=-1,
    )
    pooled = []
    off = 0
    for i, k in enumerate(_NSAMPLE_LIST):
        x = g[:, :, off:off + k, :]
        off += k
        for li, p in enumerate(mlp_params[i]):
            wt = p["W"].T
            if li == 0:
                wt = jnp.concatenate(
                    [wt, jnp.zeros((_CPAD - 9, wt.shape[1]), jnp.float32)], axis=0
                )
                z, stats = _run_mm(x, wt, sub=sub)
            else:
                z, stats = _run_mm(x, wt)
            last = li == len(mlp_params[i]) - 1
            x = _run_bn(
                z, stats, p["gamma"], p["beta"], p["a"], _B * _NPOINT * k, pool=last
            )
            if not last:
                pass
        pooled.append(x)
    x = jnp.concatenate(pooled, axis=-1)[:, :, None, :]
    for p in final_params:
        z, stats = _run_mm(x, p["W"].T)
        y = _run_bn(z, stats, p["gamma"], p["beta"], p["a"], _B * _NPOINT, pool=True)
        x = y[:, :, None, :]
    out = jnp.transpose(y, (0, 2, 1))
    return new_xyz, out
